# Initial kernel scaffold; baseline (speedup 1.0000x reference)
import jax
import jax.numpy as jnp
from jax.experimental import pallas as pl


def kernel(hidden_states, rms_weight, router_w, w_gate, w_up, w_down):
    raise NotImplementedError("write your pallas kernel here")


# speedup vs baseline: 1.7002x; 1.7002x over previous
import functools

import jax
import jax.numpy as jnp
from jax.experimental import pallas as pl

E = 8
TOPK = 2
D = 1024
DI = 512
EPS = 1e-06

TILE = 256


def _moe_body(x_ref, rmsw_ref, rw_ref, wg_ref, wu_ref, wd_ref,
              out_ref, logits_ref):
    x = x_ref[...]
    var = jnp.mean(x * x, axis=1, keepdims=True)
    xn = x * jax.lax.rsqrt(var + EPS) * rmsw_ref[...]

    logits = jax.lax.dot_general(
        xn, rw_ref[...], (((1,), (1,)), ((), ())),
        preferred_element_type=jnp.float32)
    logits_ref[...] = logits

    m = jnp.max(logits, axis=1, keepdims=True)
    ex = jnp.exp(logits - m)
    w = ex / jnp.sum(ex, axis=1, keepdims=True)

    iota = jax.lax.broadcasted_iota(jnp.int32, (TILE, E), 1)
    m1 = jnp.max(w, axis=1, keepdims=True)
    i1 = jnp.min(jnp.where(w == m1, iota, E), axis=1, keepdims=True)
    oh1 = iota == i1
    w2 = jnp.where(oh1, -jnp.inf, w)
    m2 = jnp.max(w2, axis=1, keepdims=True)
    i2 = jnp.min(jnp.where(w2 == m2, iota, E), axis=1, keepdims=True)
    oh2 = iota == i2
    cw = jnp.where(oh1 | oh2, w, 0.0)

    xb = xn.astype(jnp.bfloat16)
    acc = jnp.zeros((TILE, D), jnp.float32)
    for e in range(E):
        g = jnp.dot(xb, wg_ref[e], preferred_element_type=jnp.float32)
        u = jnp.dot(xb, wu_ref[e], preferred_element_type=jnp.float32)
        h = (g * jax.nn.sigmoid(g)) * u
        d = jnp.dot(h.astype(jnp.bfloat16), wd_ref[e],
                    preferred_element_type=jnp.float32)
        acc = acc + cw[:, e:e + 1] * d
    out_ref[...] = acc


@functools.partial(jax.jit, static_argnames=())
def kernel(hidden_states, rms_weight, router_w, w_gate, w_up, w_down):
    shape = hidden_states.shape
    T = shape[0] * shape[1]
    x = hidden_states.reshape(T, D).astype(jnp.float32)
    wg = w_gate.astype(jnp.bfloat16).transpose(0, 2, 1)
    wu = w_up.astype(jnp.bfloat16).transpose(0, 2, 1)
    wd = w_down.astype(jnp.bfloat16).transpose(0, 2, 1)

    out, logits = pl.pallas_call(
        _moe_body,
        grid=(T // TILE,),
        in_specs=[
            pl.BlockSpec((TILE, D), lambda i: (i, 0)),
            pl.BlockSpec((1, D), lambda i: (0, 0)),
            pl.BlockSpec((E, D), lambda i: (0, 0)),
            pl.BlockSpec((E, D, DI), lambda i: (0, 0, 0)),
            pl.BlockSpec((E, D, DI), lambda i: (0, 0, 0)),
            pl.BlockSpec((E, DI, D), lambda i: (0, 0, 0)),
        ],
        out_specs=[
            pl.BlockSpec((TILE, D), lambda i: (i, 0)),
            pl.BlockSpec((TILE, E), lambda i: (i, 0)),
        ],
        out_shape=[
            jax.ShapeDtypeStruct((T, D), jnp.float32),
            jax.ShapeDtypeStruct((T, E), jnp.float32),
        ],
    )(x, rms_weight.reshape(1, D), router_w, wg, wu, wd)
    return out.reshape(shape), logits


# speedup vs baseline: 1.9840x; 1.1670x over previous
import functools

import jax
import jax.numpy as jnp
from jax.experimental import pallas as pl

E = 8
TOPK = 2
D = 1024
DI = 512
EPS = 1e-06

TILE = 256


def _moe_body(x_ref, rmsw_ref, rw_ref, wg_ref, wu_ref, wd_ref,
              out_ref, logits_ref):
    x = x_ref[...]
    var = jnp.mean(x * x, axis=1, keepdims=True)
    xn = x * jax.lax.rsqrt(var + EPS) * rmsw_ref[...]

    logits = jax.lax.dot_general(
        xn, rw_ref[...], (((1,), (1,)), ((), ())),
        preferred_element_type=jnp.float32)
    logits_ref[...] = logits

    m = jnp.max(logits, axis=1, keepdims=True)
    ex = jnp.exp(logits - m)
    w = ex / jnp.sum(ex, axis=1, keepdims=True)

    iota = jax.lax.broadcasted_iota(jnp.int32, (TILE, E), 1)
    m1 = jnp.max(w, axis=1, keepdims=True)
    i1 = jnp.min(jnp.where(w == m1, iota, E), axis=1, keepdims=True)
    oh1 = iota == i1
    w2 = jnp.where(oh1, -jnp.inf, w)
    m2 = jnp.max(w2, axis=1, keepdims=True)
    i2 = jnp.min(jnp.where(w2 == m2, iota, E), axis=1, keepdims=True)
    oh2 = iota == i2
    cw = jnp.where(oh1 | oh2, w, 0.0)

    xb = xn.astype(jnp.bfloat16)
    tdot = lambda a, b: jax.lax.dot_general(
        a, b, (((1,), (1,)), ((), ())), preferred_element_type=jnp.float32)
    acc = jnp.zeros((TILE, D), jnp.float32)
    for e in range(E):
        g = tdot(xb, wg_ref[e])
        u = tdot(xb, wu_ref[e])
        h = (g * jax.nn.sigmoid(g)) * u
        d = tdot(h.astype(jnp.bfloat16), wd_ref[e])
        acc = acc + cw[:, e:e + 1] * d
    out_ref[...] = acc


@functools.partial(jax.jit, static_argnames=())
def kernel(hidden_states, rms_weight, router_w, w_gate, w_up, w_down):
    shape = hidden_states.shape
    T = shape[0] * shape[1]
    x = hidden_states.reshape(T, D).astype(jnp.float32)
    wg = w_gate.astype(jnp.bfloat16)
    wu = w_up.astype(jnp.bfloat16)
    wd = w_down.astype(jnp.bfloat16)

    out, logits = pl.pallas_call(
        _moe_body,
        grid=(T // TILE,),
        in_specs=[
            pl.BlockSpec((TILE, D), lambda i: (i, 0)),
            pl.BlockSpec((1, D), lambda i: (0, 0)),
            pl.BlockSpec((E, D), lambda i: (0, 0)),
            pl.BlockSpec((E, DI, D), lambda i: (0, 0, 0)),
            pl.BlockSpec((E, DI, D), lambda i: (0, 0, 0)),
            pl.BlockSpec((E, D, DI), lambda i: (0, 0, 0)),
        ],
        out_specs=[
            pl.BlockSpec((TILE, D), lambda i: (i, 0)),
            pl.BlockSpec((TILE, E), lambda i: (i, 0)),
        ],
        out_shape=[
            jax.ShapeDtypeStruct((T, D), jnp.float32),
            jax.ShapeDtypeStruct((T, E), jnp.float32),
        ],
    )(x, rms_weight.reshape(1, D), router_w, wg, wu, wd)
    return out.reshape(shape), logits


# speedup vs baseline: 2.4874x; 1.2537x over previous
import jax
import jax.numpy as jnp
from jax.experimental import pallas as pl
from jax.experimental.pallas import tpu as pltpu

E = 8
TOPK = 2
D = 1024
DI = 512
EPS = 1e-06

CHUNK = 512


def _tdot(a, b):
    return jax.lax.dot_general(
        a, b, (((1,), (1,)), ((), ())), preferred_element_type=jnp.float32)


def _moe_body(x_ref, rmsw_ref, rw_ref, wg_ref, wu_ref, wd_ref,
              out_ref, logits_ref,
              xn_ref, cw_ref, wgb_ref, wub_ref, wdb_ref):
    e = pl.program_id(0)
    T = x_ref.shape[0]

    @pl.when(e == 0)
    def _router():
        x = x_ref[...]
        var = jnp.mean(x * x, axis=1, keepdims=True)
        xn = x * jax.lax.rsqrt(var + EPS) * rmsw_ref[...]
        logits = jax.lax.dot_general(
            xn, rw_ref[...], (((1,), (1,)), ((), ())),
            preferred_element_type=jnp.float32)
        logits_ref[...] = logits

        m = jnp.max(logits, axis=1, keepdims=True)
        ex = jnp.exp(logits - m)
        w = ex / jnp.sum(ex, axis=1, keepdims=True)
        iota = jax.lax.broadcasted_iota(jnp.int32, (T, E), 1)
        m1 = jnp.max(w, axis=1, keepdims=True)
        i1 = jnp.min(jnp.where(w == m1, iota, E), axis=1, keepdims=True)
        oh1 = iota == i1
        w2 = jnp.where(oh1, -jnp.inf, w)
        m2 = jnp.max(w2, axis=1, keepdims=True)
        i2 = jnp.min(jnp.where(w2 == m2, iota, E), axis=1, keepdims=True)
        cw_ref[...] = jnp.where(oh1 | (iota == i2), w, 0.0)

        xn_ref[...] = xn.astype(jnp.bfloat16)
        out_ref[...] = jnp.zeros((T, D), jnp.float32)

    wgb_ref[...] = wg_ref[0].astype(jnp.bfloat16)
    wub_ref[...] = wu_ref[0].astype(jnp.bfloat16)

    oh_e = (jax.lax.broadcasted_iota(jnp.int32, (E, 1), 0) == e
            ).astype(jnp.float32)

    for c in range(T // CHUNK):
        sl = pl.ds(c * CHUNK, CHUNK)
        xb = xn_ref[sl, :]
        g = _tdot(xb, wgb_ref[...])
        u = _tdot(xb, wub_ref[...])
        if c == 0:
            wdb_ref[...] = wd_ref[0].astype(jnp.bfloat16)
        h = ((g * jax.nn.sigmoid(g)) * u).astype(jnp.bfloat16)
        d = _tdot(h, wdb_ref[...])
        wcol = jax.lax.dot_general(
            cw_ref[sl, :], oh_e, (((1,), (0,)), ((), ())),
            preferred_element_type=jnp.float32)
        out_ref[sl, :] += wcol * d


def kernel(hidden_states, rms_weight, router_w, w_gate, w_up, w_down):
    shape = hidden_states.shape
    T = shape[0] * shape[1]
    x = hidden_states.reshape(T, D).astype(jnp.float32)

    out, logits = pl.pallas_call(
        _moe_body,
        grid=(E,),
        in_specs=[
            pl.BlockSpec((T, D), lambda e: (0, 0)),
            pl.BlockSpec((1, D), lambda e: (0, 0)),
            pl.BlockSpec((E, D), lambda e: (0, 0)),
            pl.BlockSpec((1, DI, D), lambda e: (e, 0, 0)),
            pl.BlockSpec((1, DI, D), lambda e: (e, 0, 0)),
            pl.BlockSpec((1, D, DI), lambda e: (e, 0, 0)),
        ],
        out_specs=[
            pl.BlockSpec((T, D), lambda e: (0, 0)),
            pl.BlockSpec((T, E), lambda e: (0, 0)),
        ],
        out_shape=[
            jax.ShapeDtypeStruct((T, D), jnp.float32),
            jax.ShapeDtypeStruct((T, E), jnp.float32),
        ],
        scratch_shapes=[
            pltpu.VMEM((T, D), jnp.bfloat16),
            pltpu.VMEM((T, E), jnp.float32),
            pltpu.VMEM((DI, D), jnp.bfloat16),
            pltpu.VMEM((DI, D), jnp.bfloat16),
            pltpu.VMEM((D, DI), jnp.bfloat16),
        ],
    )(x, rms_weight.reshape(1, D), router_w, w_gate, w_up, w_down)
    return out.reshape(shape), logits


# speedup vs baseline: 2.7058x; 1.0878x over previous
import jax
import jax.numpy as jnp
from jax.experimental import pallas as pl
from jax.experimental.pallas import tpu as pltpu

E = 8
TOPK = 2
D = 1024
DI = 512
EPS = 1e-06

CHUNK = 1024


def _tdot(a, b):
    return jax.lax.dot_general(
        a, b, (((1,), (1,)), ((), ())), preferred_element_type=jnp.float32)


def _moe_body(x_ref, rmsw_ref, rw_ref, wg_ref, wu_ref, wd_ref,
              out_ref, logits_ref,
              xn_ref, cw_ref, wgb_ref, wub_ref, wdb_ref):
    e = pl.program_id(0)
    T = x_ref.shape[0]

    @pl.when(e == 0)
    def _router():
        x = x_ref[...]
        var = jnp.mean(x * x, axis=1, keepdims=True)
        xn = x * jax.lax.rsqrt(var + EPS) * rmsw_ref[...]
        logits = jax.lax.dot_general(
            xn, rw_ref[...], (((1,), (1,)), ((), ())),
            preferred_element_type=jnp.float32)
        logits_ref[...] = logits

        m = jnp.max(logits, axis=1, keepdims=True)
        ex = jnp.exp(logits - m)
        w = ex / jnp.sum(ex, axis=1, keepdims=True)
        iota = jax.lax.broadcasted_iota(jnp.int32, (T, E), 1)
        m1 = jnp.max(w, axis=1, keepdims=True)
        i1 = jnp.min(jnp.where(w == m1, iota, E), axis=1, keepdims=True)
        oh1 = iota == i1
        w2 = jnp.where(oh1, -jnp.inf, w)
        m2 = jnp.max(w2, axis=1, keepdims=True)
        i2 = jnp.min(jnp.where(w2 == m2, iota, E), axis=1, keepdims=True)
        cw_ref[...] = jnp.where(oh1 | (iota == i2), w, 0.0)

        xn_ref[...] = xn.astype(jnp.bfloat16)
        out_ref[...] = jnp.zeros((T, D), jnp.float32)

    wgb_ref[...] = wg_ref[0].astype(jnp.bfloat16)
    wub_ref[...] = wu_ref[0].astype(jnp.bfloat16)

    oh_e = (jax.lax.broadcasted_iota(jnp.int32, (E, 1), 0) == e
            ).astype(jnp.float32)

    for c in range(T // CHUNK):
        sl = pl.ds(c * CHUNK, CHUNK)
        xb = xn_ref[sl, :]
        g = _tdot(xb, wgb_ref[...])
        u = _tdot(xb, wub_ref[...])
        if c == 0:
            wdb_ref[...] = wd_ref[0].astype(jnp.bfloat16)
        wcol = jax.lax.dot_general(
            cw_ref[sl, :], oh_e, (((1,), (0,)), ((), ())),
            preferred_element_type=jnp.float32)
        h = ((g * jax.nn.sigmoid(g)) * u * wcol).astype(jnp.bfloat16)
        out_ref[sl, :] += _tdot(h, wdb_ref[...])


def kernel(hidden_states, rms_weight, router_w, w_gate, w_up, w_down):
    shape = hidden_states.shape
    T = shape[0] * shape[1]
    x = hidden_states.reshape(T, D).astype(jnp.float32)

    out, logits = pl.pallas_call(
        _moe_body,
        grid=(E,),
        in_specs=[
            pl.BlockSpec((T, D), lambda e: (0, 0)),
            pl.BlockSpec((1, D), lambda e: (0, 0)),
            pl.BlockSpec((E, D), lambda e: (0, 0)),
            pl.BlockSpec((1, DI, D), lambda e: (e, 0, 0)),
            pl.BlockSpec((1, DI, D), lambda e: (e, 0, 0)),
            pl.BlockSpec((1, D, DI), lambda e: (e, 0, 0)),
        ],
        out_specs=[
            pl.BlockSpec((T, D), lambda e: (0, 0)),
            pl.BlockSpec((T, E), lambda e: (0, 0)),
        ],
        out_shape=[
            jax.ShapeDtypeStruct((T, D), jnp.float32),
            jax.ShapeDtypeStruct((T, E), jnp.float32),
        ],
        scratch_shapes=[
            pltpu.VMEM((T, D), jnp.bfloat16),
            pltpu.VMEM((T, E), jnp.float32),
            pltpu.VMEM((DI, D), jnp.bfloat16),
            pltpu.VMEM((DI, D), jnp.bfloat16),
            pltpu.VMEM((D, DI), jnp.bfloat16),
        ],
    )(x, rms_weight.reshape(1, D), router_w, w_gate, w_up, w_down)
    return out.reshape(shape), logits
